# Initial kernel scaffold; baseline (speedup 1.0000x reference)
import jax
import jax.numpy as jnp
from jax.experimental import pallas as pl


def kernel(x, industry_edge_index, universe_edge_index, bn_g, bn_b, enc_W, enc_b, gi_W, gi_as, gi_ad, gi_b, gu_W, gu_as, gu_ad, gu_b, d1_W, d1_b, d2_W, d2_b, fa_W, fa_b):
    raise NotImplementedError("write your pallas kernel here")


# speedup vs baseline: 31.8955x; 31.8955x over previous
import functools

import jax
import jax.numpy as jnp
from jax import lax
from jax.experimental import pallas as pl
from jax.experimental.pallas import tpu as pltpu
from jax.experimental.pallas import tpu_sc as plsc

N = 50000
F_IN = 128
HID = 32
HEADS = 2
E = 800000

NS = 16
L = 16
CH = 128
N_P = 51200
ROWS_PT = N_P // NS
E_P = 16 * 416 * CH
EPT = E_P // NS
NCHUNKS = EPT // CH


def _gat_body(src_hbm, dst_hbm, asrc_hbm, adst_hbm, h_hbm, out_hbm, den_hbm,
              sidx, didx, rows, wbuf, av, bv, zb, zbd,
              asrc_s, adst_s, out_s, den_s, sem, sem2, sem3):
    c = lax.axis_index("c")
    s = lax.axis_index("s")

    row0 = s * ROWS_PT
    pltpu.sync_copy(asrc_hbm.at[c].at[pl.ds(row0, ROWS_PT)],
                    asrc_s.at[pl.ds(row0, ROWS_PT)])
    pltpu.sync_copy(adst_hbm.at[c].at[pl.ds(row0, ROWS_PT)],
                    adst_s.at[pl.ds(row0, ROWS_PT)])
    zv = jnp.zeros((L,), jnp.float32)
    for i in range(64):
        zb[i, pl.ds(0, L)] = zv
        zb[i, pl.ds(L, L)] = zv
    for i in range(4):
        zbd[pl.ds(i * L, L)] = zv
    for k in range(ROWS_PT // 64):
        pltpu.sync_copy(zb, out_s.at[pl.ds(row0 + k * 64, 64)])
        pltpu.sync_copy(zbd, den_s.at[pl.ds(row0 + k * 64, 64)])
    plsc.subcore_barrier()

    e0 = s * EPT
    lane = lax.iota(jnp.int32, L)

    def chunk(g, _):
        base = e0 + g * CH
        pltpu.sync_copy(src_hbm.at[pl.ds(base, CH)], sidx)
        pltpu.sync_copy(dst_hbm.at[pl.ds(base, CH)], didx)
        cp_rows = pltpu.async_copy(h_hbm.at[c].at[sidx], rows, sem)
        cp_a = pltpu.async_copy(asrc_s.at[sidx], av, sem2)
        cp_b = pltpu.async_copy(adst_s.at[didx], bv, sem3)
        cp_a.wait()
        cp_b.wait()
        cp_rows.wait()
        for j in range(CH // L):
            a = av[pl.ds(j * L, L)] + bv[pl.ds(j * L, L)]
            w = jnp.exp(jnp.where(a > 0, a, 0.2 * a))
            wbuf[pl.ds(j * L, L)] = w
            rid = lane + j * L
            for col in range(HID):
                cid = jnp.full((L,), col, jnp.int32)
                v = plsc.load_gather(rows, [rid, cid]) * w
                plsc.store_scatter(rows, [rid, cid], v)
        pltpu.sync_copy(rows, out_s.at[didx], add=True)
        pltpu.sync_copy(wbuf, den_s.at[didx], add=True)
        return ()

    lax.fori_loop(0, NCHUNKS, chunk, (), unroll=False)
    plsc.subcore_barrier()

    pltpu.sync_copy(out_s.at[pl.ds(row0, ROWS_PT)], out_hbm.at[c].at[pl.ds(row0, ROWS_PT)])
    pltpu.sync_copy(den_s.at[pl.ds(row0, ROWS_PT)], den_hbm.at[c].at[pl.ds(row0, ROWS_PT)])


_gat_sc = functools.partial(
    pl.kernel,
    out_type=(
        jax.ShapeDtypeStruct((HEADS, N_P, HID), jnp.float32),
        jax.ShapeDtypeStruct((HEADS, N_P), jnp.float32),
    ),
    mesh=plsc.VectorSubcoreMesh(core_axis_name="c", subcore_axis_name="s"),
    compiler_params=pltpu.CompilerParams(
        use_tc_tiling_on_sc=False, needs_layout_passes=False),
    scratch_types=(
        pltpu.VMEM((CH,), jnp.int32),
        pltpu.VMEM((CH,), jnp.int32),
        pltpu.VMEM((CH, HID), jnp.float32),
        pltpu.VMEM((CH,), jnp.float32),
        pltpu.VMEM((CH,), jnp.float32),
        pltpu.VMEM((CH,), jnp.float32),
        pltpu.VMEM((64, HID), jnp.float32),
        pltpu.VMEM((64,), jnp.float32),
        pltpu.VMEM_SHARED((N_P,), jnp.float32),
        pltpu.VMEM_SHARED((N_P,), jnp.float32),
        pltpu.VMEM_SHARED((N_P, HID), jnp.float32),
        pltpu.VMEM_SHARED((N_P,), jnp.float32),
        pltpu.SemaphoreType.DMA,
        pltpu.SemaphoreType.DMA,
        pltpu.SemaphoreType.DMA,
    ),
)(_gat_body)


def _gat_conv_sc(x_nodes, edge_index, W, att_src, att_dst, bias):
    src = edge_index[0]
    dst = edge_index[1]
    loops = jnp.arange(N, dtype=src.dtype)
    pad = jnp.full((E_P - E - N,), N, src.dtype)
    src_p = jnp.concatenate([src, loops, pad])
    dst_p = jnp.concatenate([dst, loops, pad])

    h = (x_nodes @ W).reshape(N, HEADS, HID)
    a_src = jnp.sum(h * att_src, axis=-1)
    a_dst = jnp.sum(h * att_dst, axis=-1)
    zpadN = jnp.zeros((HEADS, N_P - N), jnp.float32)
    asrc_p = jnp.concatenate([a_src.T, zpadN], axis=1)
    adst_p = jnp.concatenate([a_dst.T, zpadN], axis=1)
    h_p = jnp.concatenate(
        [h.transpose(1, 0, 2), jnp.zeros((HEADS, N_P - N, HID), jnp.float32)], axis=1)

    out, den = _gat_sc(src_p, dst_p, asrc_p, adst_p, h_p)
    out = out[:, :N, :] / (den[:, :N, None] + 1e-16)
    return out.mean(axis=0) + bias


def kernel(x, industry_edge_index, universe_edge_index, bn_g, bn_b, enc_W, enc_b,
           gi_W, gi_as, gi_ad, gi_b, gu_W, gu_as, gu_ad, gu_b,
           d1_W, d1_b, d2_W, d2_b, fa_W, fa_b):
    mean = jnp.mean(x, axis=0)
    var = jnp.mean((x - mean) ** 2, axis=0)
    x_norm = (x - mean) / jnp.sqrt(var + 1e-5) * bn_g + bn_b
    C = jax.nn.relu(x_norm @ enc_W + enc_b)
    H_I = jax.nn.elu(_gat_conv_sc(C, industry_edge_index, gi_W, gi_as, gi_ad, gi_b))
    C_I = C - H_I
    H_U = jax.nn.elu(_gat_conv_sc(C_I, universe_edge_index, gu_W, gu_as, gu_ad, gu_b))
    C_U = C_I - H_U
    hierarchical = jnp.concatenate([C, C_I, C_U], axis=-1)
    deep_factor = jax.nn.relu(hierarchical @ d1_W + d1_b) @ d2_W + d2_b
    U = jax.nn.leaky_relu(x @ fa_W + fa_b, negative_slope=0.2)
    attn_weights = jax.nn.softmax(U, axis=-1)
    return (deep_factor, attn_weights, C, C_I, C_U, H_I, H_U)


# speedup vs baseline: 38.1646x; 1.1966x over previous
import functools

import jax
import jax.numpy as jnp
from jax import lax
from jax.experimental import pallas as pl
from jax.experimental.pallas import tpu as pltpu
from jax.experimental.pallas import tpu_sc as plsc

N = 50000
F_IN = 128
HID = 32
HEADS = 2
E = 800000

NS = 16
L = 16
CH = 128
N_P = 51200
ROWS_PT = N_P // NS
E_P = 16 * 416 * CH
EPT = E_P // NS
NCHUNKS = EPT // CH

R = 1000
GRID = N // R


def _gat_body(src_hbm, dst_hbm, asrc_hbm, adst_hbm, h_hbm, out_hbm,
              sidx0, sidx1, sidx2, sidx3, didx0, didx1, didx2, didx3,
              rows0, rows1, wbuf0, wbuf1, av0, av1, bv0, bv1, zb, zbd,
              asrc_s, adst_s, out_s, den_s,
              semi0, semi1, semi2, semi3, semr0, semr1,
              semab0, semab1, semsc0, semsc1):
    c = lax.axis_index("c")
    s = lax.axis_index("s")
    sidxl = (sidx0, sidx1, sidx2, sidx3)
    didxl = (didx0, didx1, didx2, didx3)
    rowsl = (rows0, rows1)
    wbufl = (wbuf0, wbuf1)
    avl = (av0, av1)
    bvl = (bv0, bv1)
    semi = (semi0, semi1, semi2, semi3)
    semr = (semr0, semr1)
    semab = (semab0, semab1)
    semsc = (semsc0, semsc1)

    row0 = s * ROWS_PT
    pltpu.sync_copy(asrc_hbm.at[c].at[pl.ds(row0, ROWS_PT)],
                    asrc_s.at[pl.ds(row0, ROWS_PT)])
    pltpu.sync_copy(adst_hbm.at[c].at[pl.ds(row0, ROWS_PT)],
                    adst_s.at[pl.ds(row0, ROWS_PT)])
    zv = jnp.zeros((L,), jnp.float32)
    for i in range(64):
        zb[i, pl.ds(0, L)] = zv
        zb[i, pl.ds(L, L)] = zv
    for i in range(4):
        zbd[pl.ds(i * L, L)] = zv
    for k in range(ROWS_PT // 64):
        pltpu.sync_copy(zb, out_s.at[pl.ds(row0 + k * 64, 64)])
        pltpu.sync_copy(zbd, den_s.at[pl.ds(row0 + k * 64, 64)])
    plsc.subcore_barrier()

    e0 = s * EPT
    lane = lax.iota(jnp.int32, L)

    def start_idx(cg, s4):
        base = e0 + cg * CH
        pltpu.async_copy(src_hbm.at[pl.ds(base, CH)], sidxl[s4], semi[s4])
        pltpu.async_copy(dst_hbm.at[pl.ds(base, CH)], didxl[s4], semi[s4])

    def wait_idx(cg, s4):
        base = e0 + cg * CH
        pltpu.make_async_copy(src_hbm.at[pl.ds(base, CH)], sidxl[s4], semi[s4]).wait()
        pltpu.make_async_copy(dst_hbm.at[pl.ds(base, CH)], didxl[s4], semi[s4]).wait()

    def start_gathers(s4, s2):
        pltpu.async_copy(h_hbm.at[c].at[sidxl[s4]], rowsl[s2], semr[s2])
        pltpu.async_copy(asrc_s.at[sidxl[s4]], avl[s2], semab[s2])
        pltpu.async_copy(adst_s.at[didxl[s4]], bvl[s2], semab[s2])

    def process(s4, s2):
        pltpu.make_async_copy(asrc_s.at[sidxl[s4]], avl[s2], semab[s2]).wait()
        pltpu.make_async_copy(adst_s.at[didxl[s4]], bvl[s2], semab[s2]).wait()
        pltpu.make_async_copy(h_hbm.at[c].at[sidxl[s4]], rowsl[s2], semr[s2]).wait()
        rows = rowsl[s2]
        wbuf = wbufl[s2]
        for j in range(CH // L):
            a = avl[s2][pl.ds(j * L, L)] + bvl[s2][pl.ds(j * L, L)]
            w = jnp.exp(jnp.where(a > 0, a, 0.2 * a))
            wbuf[pl.ds(j * L, L)] = w
            rid = lane + j * L
            for col in range(HID):
                cid = jnp.full((L,), col, jnp.int32)
                v = plsc.load_gather(rows, [rid, cid]) * w
                plsc.store_scatter(rows, [rid, cid], v)
        pltpu.async_copy(rows, out_s.at[didxl[s4]], semsc[s2], add=True)
        pltpu.async_copy(wbuf, den_s.at[didxl[s4]], semsc[s2], add=True)

    def wait_scatter(s4, s2):
        pltpu.make_async_copy(rowsl[s2], out_s.at[didxl[s4]], semsc[s2]).wait()
        pltpu.make_async_copy(wbufl[s2], den_s.at[didxl[s4]], semsc[s2]).wait()

    start_idx(0, 0)

    def outer(i, _):
        for k in range(4):
            cg = i * 4 + k

            @pl.when(jnp.logical_and(cg >= 2, cg < NCHUNKS + 2))
            def _():
                wait_scatter((k + 2) % 4, k % 2)

            @pl.when(cg < NCHUNKS)
            def _():
                wait_idx(cg, k)
                start_gathers(k, k % 2)

            @pl.when(cg + 1 < NCHUNKS)
            def _():
                start_idx(cg + 1, (k + 1) % 4)

            @pl.when(jnp.logical_and(cg >= 1, cg < NCHUNKS + 1))
            def _():
                process((k + 3) % 4, (k + 1) % 2)
        return ()

    lax.fori_loop(0, NCHUNKS // 4 + 1, outer, (), unroll=False)
    plsc.subcore_barrier()

    def divrow(k, _):
        r = row0 + k * 64
        pltpu.sync_copy(out_s.at[pl.ds(r, 64)], zb)
        pltpu.sync_copy(den_s.at[pl.ds(r, 64)], zbd)
        for j in range(4):
            rec = 1.0 / (zbd[pl.ds(j * L, L)] + 1e-16)
            rid = lane + j * L
            for col in range(HID):
                cid = jnp.full((L,), col, jnp.int32)
                v = plsc.load_gather(zb, [rid, cid]) * rec
                plsc.store_scatter(zb, [rid, cid], v)
        pltpu.sync_copy(zb, out_hbm.at[c].at[pl.ds(r, 64)])
        return ()

    lax.fori_loop(0, ROWS_PT // 64, divrow, (), unroll=False)


_gat_sc = functools.partial(
    pl.kernel,
    out_type=jax.ShapeDtypeStruct((HEADS, N_P, HID), jnp.float32),
    mesh=plsc.VectorSubcoreMesh(core_axis_name="c", subcore_axis_name="s"),
    compiler_params=pltpu.CompilerParams(
        use_tc_tiling_on_sc=False, needs_layout_passes=False),
    scratch_types=(
        tuple([pltpu.VMEM((CH,), jnp.int32)] * 8)
        + tuple([pltpu.VMEM((CH, HID), jnp.float32)] * 2)
        + tuple([pltpu.VMEM((CH,), jnp.float32)] * 6)
        + (
            pltpu.VMEM((64, HID), jnp.float32),
            pltpu.VMEM((64,), jnp.float32),
            pltpu.VMEM_SHARED((N_P,), jnp.float32),
            pltpu.VMEM_SHARED((N_P,), jnp.float32),
            pltpu.VMEM_SHARED((N_P, HID), jnp.float32),
            pltpu.VMEM_SHARED((N_P,), jnp.float32),
        )
        + tuple([pltpu.SemaphoreType.DMA] * 10)
    ),
)(_gat_body)


def _gat_conv_sc(edge_index, aout, h_heads):
    src = edge_index[0]
    dst = edge_index[1]
    loops = jnp.arange(N, dtype=src.dtype)
    pad = jnp.full((E_P - E - N,), N, src.dtype)
    src_p = jnp.concatenate([src, loops, pad])
    dst_p = jnp.concatenate([dst, loops, pad])
    ap = jnp.pad(aout, ((0, N_P - N), (0, 0)))
    asrc_p = ap[:, 0:2].T
    adst_p = ap[:, 2:4].T
    return _gat_sc(src_p, dst_p, asrc_p, adst_p, h_heads)


def _stats_body(x_ref, s1_ref, s2_ref):
    i = pl.program_id(0)
    xb = x_ref[...].reshape(R // 8, 8, F_IN)
    ps = jnp.sum(xb, axis=0)
    ps2 = jnp.sum(xb * xb, axis=0)

    @pl.when(i == 0)
    def _():
        s1_ref[...] = ps
        s2_ref[...] = ps2

    @pl.when(i > 0)
    def _():
        s1_ref[...] += ps
        s2_ref[...] += ps2


_stats_call = pl.pallas_call(
    _stats_body,
    grid=(GRID,),
    in_specs=[pl.BlockSpec((R, F_IN), lambda i: (i, 0))],
    out_specs=(pl.BlockSpec((8, F_IN), lambda i: (0, 0)),
               pl.BlockSpec((8, F_IN), lambda i: (0, 0))),
    out_shape=(jax.ShapeDtypeStruct((8, F_IN), jnp.float32),
               jax.ShapeDtypeStruct((8, F_IN), jnp.float32)),
)


def _proj_tail(h, as_ref, ad_ref, h_ref, a_ref):
    h0 = h[:, :HID]
    h1 = h[:, HID:]
    h_ref[0] = h0
    h_ref[1] = h1
    as_ = as_ref[...]
    ad_ = ad_ref[...]
    a_ref[:, 0:1] = jnp.sum(h0 * as_[0:1, :], axis=1, keepdims=True)
    a_ref[:, 1:2] = jnp.sum(h1 * as_[1:2, :], axis=1, keepdims=True)
    a_ref[:, 2:3] = jnp.sum(h0 * ad_[0:1, :], axis=1, keepdims=True)
    a_ref[:, 3:4] = jnp.sum(h1 * ad_[1:2, :], axis=1, keepdims=True)


def _pre_body(x_ref, sc1_ref, sc0_ref, encW_ref, encb_ref, giW_ref,
              gias_ref, giad_ref, C_ref, h_ref, a_ref):
    xn = x_ref[...] * sc1_ref[...] + sc0_ref[...]
    C = jnp.maximum(jnp.dot(xn, encW_ref[...]) + encb_ref[...], 0.0)
    C_ref[...] = C
    _proj_tail(jnp.dot(C, giW_ref[...]), gias_ref, giad_ref, h_ref, a_ref)


_pre_call = pl.pallas_call(
    _pre_body,
    grid=(GRID,),
    in_specs=[
        pl.BlockSpec((R, F_IN), lambda i: (i, 0)),
        pl.BlockSpec((1, F_IN), lambda i: (0, 0)),
        pl.BlockSpec((1, F_IN), lambda i: (0, 0)),
        pl.BlockSpec((F_IN, HID), lambda i: (0, 0)),
        pl.BlockSpec((1, HID), lambda i: (0, 0)),
        pl.BlockSpec((HID, HEADS * HID), lambda i: (0, 0)),
        pl.BlockSpec((HEADS, HID), lambda i: (0, 0)),
        pl.BlockSpec((HEADS, HID), lambda i: (0, 0)),
    ],
    out_specs=(pl.BlockSpec((R, HID), lambda i: (i, 0)),
               pl.BlockSpec((HEADS, R, HID), lambda i: (0, i, 0)),
               pl.BlockSpec((R, 4), lambda i: (i, 0))),
    out_shape=(jax.ShapeDtypeStruct((N, HID), jnp.float32),
               jax.ShapeDtypeStruct((HEADS, N_P, HID), jnp.float32),
               jax.ShapeDtypeStruct((N, 4), jnp.float32)),
)


def _attn_body(x_ref, faW_ref, fab_ref, attn_ref):
    u = jnp.dot(x_ref[...], faW_ref[...]) + fab_ref[...]
    u = jnp.where(u > 0, u, 0.2 * u)
    m = jnp.max(u, axis=1, keepdims=True)
    e = jnp.exp(u - m)
    attn_ref[...] = e / jnp.sum(e, axis=1, keepdims=True)


_attn_call = pl.pallas_call(
    _attn_body,
    grid=(GRID,),
    in_specs=[
        pl.BlockSpec((R, F_IN), lambda i: (i, 0)),
        pl.BlockSpec((F_IN, F_IN), lambda i: (0, 0)),
        pl.BlockSpec((1, F_IN), lambda i: (0, 0)),
    ],
    out_specs=pl.BlockSpec((R, F_IN), lambda i: (i, 0)),
    out_shape=jax.ShapeDtypeStruct((N, F_IN), jnp.float32),
)


def _elu_mean(Hh_ref, b_ref):
    hm = (Hh_ref[0] + Hh_ref[1]) * 0.5 + b_ref[...]
    return jnp.where(hm > 0, hm, jnp.exp(hm) - 1.0)


def _mid_body(C_ref, Hh_ref, gib_ref, guW_ref, guas_ref, guad_ref,
              HI_ref, CI_ref, h_ref, a_ref):
    HI = _elu_mean(Hh_ref, gib_ref)
    HI_ref[...] = HI
    CI = C_ref[...] - HI
    CI_ref[...] = CI
    _proj_tail(jnp.dot(CI, guW_ref[...]), guas_ref, guad_ref, h_ref, a_ref)


_mid_call = pl.pallas_call(
    _mid_body,
    grid=(GRID,),
    in_specs=[
        pl.BlockSpec((R, HID), lambda i: (i, 0)),
        pl.BlockSpec((HEADS, R, HID), lambda i: (0, i, 0)),
        pl.BlockSpec((1, HID), lambda i: (0, 0)),
        pl.BlockSpec((HID, HEADS * HID), lambda i: (0, 0)),
        pl.BlockSpec((HEADS, HID), lambda i: (0, 0)),
        pl.BlockSpec((HEADS, HID), lambda i: (0, 0)),
    ],
    out_specs=(pl.BlockSpec((R, HID), lambda i: (i, 0)),
               pl.BlockSpec((R, HID), lambda i: (i, 0)),
               pl.BlockSpec((HEADS, R, HID), lambda i: (0, i, 0)),
               pl.BlockSpec((R, 4), lambda i: (i, 0))),
    out_shape=(jax.ShapeDtypeStruct((N, HID), jnp.float32),
               jax.ShapeDtypeStruct((N, HID), jnp.float32),
               jax.ShapeDtypeStruct((HEADS, N_P, HID), jnp.float32),
               jax.ShapeDtypeStruct((N, 4), jnp.float32)),
)


def _fin_body(C_ref, CI_ref, Hh_ref, gub_ref, d1W_ref, d1b_ref,
              d2W_ref, d2b_ref, HU_ref, CU_ref, df_ref):
    HU = _elu_mean(Hh_ref, gub_ref)
    HU_ref[...] = HU
    CI = CI_ref[...]
    CU = CI - HU
    CU_ref[...] = CU
    d1W = d1W_ref[...]
    z = (jnp.dot(C_ref[...], d1W[0:HID, :])
         + jnp.dot(CI, d1W[HID:2 * HID, :])
         + jnp.dot(CU, d1W[2 * HID:, :]) + d1b_ref[...])
    z = jnp.maximum(z, 0.0)
    df_ref[...] = jnp.sum(z * d2W_ref[...], axis=1, keepdims=True) + d2b_ref[...]


_fin_call = pl.pallas_call(
    _fin_body,
    grid=(GRID,),
    in_specs=[
        pl.BlockSpec((R, HID), lambda i: (i, 0)),
        pl.BlockSpec((R, HID), lambda i: (i, 0)),
        pl.BlockSpec((HEADS, R, HID), lambda i: (0, i, 0)),
        pl.BlockSpec((1, HID), lambda i: (0, 0)),
        pl.BlockSpec((3 * HID, HID), lambda i: (0, 0)),
        pl.BlockSpec((1, HID), lambda i: (0, 0)),
        pl.BlockSpec((1, HID), lambda i: (0, 0)),
        pl.BlockSpec((1, 1), lambda i: (0, 0)),
    ],
    out_specs=(pl.BlockSpec((R, HID), lambda i: (i, 0)),
               pl.BlockSpec((R, HID), lambda i: (i, 0)),
               pl.BlockSpec((R, 1), lambda i: (i, 0))),
    out_shape=(jax.ShapeDtypeStruct((N, HID), jnp.float32),
               jax.ShapeDtypeStruct((N, HID), jnp.float32),
               jax.ShapeDtypeStruct((N, 1), jnp.float32)),
)


def kernel(x, industry_edge_index, universe_edge_index, bn_g, bn_b, enc_W, enc_b,
           gi_W, gi_as, gi_ad, gi_b, gu_W, gu_as, gu_ad, gu_b,
           d1_W, d1_b, d2_W, d2_b, fa_W, fa_b):
    sums, sumsq = _stats_call(x)
    tot = jnp.sum(sums, axis=0)
    tot2 = jnp.sum(sumsq, axis=0)
    mean = tot / N
    var = tot2 / N - mean * mean
    rstd = bn_g / jnp.sqrt(var + 1e-5)
    sc1 = rstd.reshape(1, F_IN)
    sc0 = (bn_b - mean * rstd).reshape(1, F_IN)

    C, h_i, a_i = _pre_call(x, sc1, sc0, enc_W, enc_b.reshape(1, HID), gi_W,
                            gi_as.reshape(HEADS, HID), gi_ad.reshape(HEADS, HID))
    attn_weights = _attn_call(x, fa_W, fa_b.reshape(1, F_IN))

    H_Ih = _gat_conv_sc(industry_edge_index, a_i, h_i)
    H_I, C_I, h_u, a_u = _mid_call(C, H_Ih, gi_b.reshape(1, HID), gu_W,
                                   gu_as.reshape(HEADS, HID),
                                   gu_ad.reshape(HEADS, HID))
    H_Uh = _gat_conv_sc(universe_edge_index, a_u, h_u)
    H_U, C_U, deep_factor = _fin_call(C, C_I, H_Uh, gu_b.reshape(1, HID),
                                      d1_W, d1_b.reshape(1, HID),
                                      d2_W.reshape(1, HID), d2_b.reshape(1, 1))
    return (deep_factor, attn_weights, C, C_I, C_U, H_I, H_U)
